# Initial kernel scaffold; baseline (speedup 1.0000x reference)
import jax
import jax.numpy as jnp
from jax.experimental import pallas as pl


def kernel(hidden_states, access_frequency, importance_score, sel_W1, sel_b1, sel_W2, sel_b2, comp_W1, comp_b1, adapt_W1, adapt_b1, decomp_W1, decomp_b1, comp_W2, comp_b2, adapt_W2, adapt_b2, decomp_W2, decomp_b2):
    raise NotImplementedError("write your pallas kernel here")


# speedup vs baseline: 1.4346x; 1.4346x over previous
import jax
import jax.numpy as jnp
from jax import lax
from jax.experimental import pallas as pl


_PREC = lax.Precision.DEFAULT


def kernel(hidden_states, access_frequency, importance_score,
           sel_W1, sel_b1, sel_W2, sel_b2,
           comp_W1, comp_b1, adapt_W1, adapt_b1, decomp_W1, decomp_b1,
           comp_W2, comp_b2, adapt_W2, adapt_b2, decomp_W2, decomp_b2):
    b, s, h = hidden_states.shape
    n = b * s
    x = hidden_states.reshape(n, h)
    af = access_frequency.reshape(n, 1)
    im = importance_score.reshape(n, 1)
    hq = sel_W1.shape[1]
    l = sel_W2.shape[1]
    sW2p = jnp.zeros((hq, 128), sel_W2.dtype).at[:, :l].set(sel_W2)
    sb2p = jnp.zeros((128,), sel_b2.dtype).at[:l].set(sel_b2)

    tb = 256
    nb = n // tb

    grid_spec = pl.GridSpec(
        grid=(nb,),
        in_specs=[
            pl.BlockSpec((tb, h), lambda i: (i, 0)),
            pl.BlockSpec((tb, 1), lambda i: (i, 0)),
            pl.BlockSpec((tb, 1), lambda i: (i, 0)),
            pl.BlockSpec((h, hq), lambda i: (0, 0)),
            pl.BlockSpec((hq,), lambda i: (0,)),
            pl.BlockSpec((hq, 128), lambda i: (0, 0)),
            pl.BlockSpec((128,), lambda i: (0,)),
            pl.BlockSpec((h, h // 2), lambda i: (0, 0)),
            pl.BlockSpec((h // 2,), lambda i: (0,)),
            pl.BlockSpec((h // 2, h // 2), lambda i: (0, 0)),
            pl.BlockSpec((h // 2,), lambda i: (0,)),
            pl.BlockSpec((h // 2, h), lambda i: (0, 0)),
            pl.BlockSpec((h,), lambda i: (0,)),
            pl.BlockSpec((h, h // 4), lambda i: (0, 0)),
            pl.BlockSpec((h // 4,), lambda i: (0,)),
            pl.BlockSpec((h // 4, h // 4), lambda i: (0, 0)),
            pl.BlockSpec((h // 4,), lambda i: (0,)),
            pl.BlockSpec((h // 4, h), lambda i: (0, 0)),
            pl.BlockSpec((h,), lambda i: (0,)),
        ],
        out_specs=pl.BlockSpec((tb, h), lambda i: (i, 0)),
    )

    def body(x_ref, af_ref, im_ref, sW1_ref, sb1_ref, sW2_ref, sb2_ref,
             cW1_ref, cb1_ref, aW1_ref, ab1_ref, dW1_ref, db1_ref,
             cW2_ref, cb2_ref, aW2_ref, ab2_ref, dW2_ref, db2_ref, out_ref):
        x_blk = x_ref[...]
        half = x_blk.shape[1] // 2
        comb_a = x_blk[:, :half] * af_ref[...]
        comb_b = x_blk[:, half:] * im_ref[...]

        def mm(a, bm):
            return jnp.dot(a, bm, precision=_PREC,
                           preferred_element_type=jnp.float32)

        hsel = mm(comb_a, sW1_ref[:half, :]) + mm(comb_b, sW1_ref[half:, :])
        hsel = jnp.maximum(hsel + sb1_ref[...], 0.0)
        logits = mm(hsel, sW2_ref[...]) + sb2_ref[...]
        l0 = logits[:, 0:1]
        l1 = logits[:, 1:2]
        l2 = logits[:, 2:3]
        sel1 = l1 > l0
        sel2 = l2 > jnp.maximum(l0, l1)

        d1 = mm(mm(mm(x_blk, cW1_ref[...]) + cb1_ref[...],
                   aW1_ref[...]) + ab1_ref[...], dW1_ref[...]) + db1_ref[...]
        d2 = mm(mm(mm(x_blk, cW2_ref[...]) + cb2_ref[...],
                   aW2_ref[...]) + ab2_ref[...], dW2_ref[...]) + db2_ref[...]
        out_ref[...] = jnp.where(sel2, d2, jnp.where(sel1, d1, x_blk))

    out = pl.pallas_call(
        body,
        grid_spec=grid_spec,
        out_shape=jax.ShapeDtypeStruct((n, h), jnp.float32),
    )(x, af, im, sel_W1, sel_b1, sW2p, sb2p,
      comp_W1, comp_b1, adapt_W1, adapt_b1, decomp_W1, decomp_b1,
      comp_W2, comp_b2, adapt_W2, adapt_b2, decomp_W2, decomp_b2)
    return out.reshape(b, s, h)
